# baseline (speedup 1.0000x reference)
import functools

import jax
import jax.numpy as jnp
from jax import lax
from jax.experimental import pallas as pl
from jax.experimental.pallas import tpu as pltpu
from jax.experimental.pallas import tpu_sc as plsc

EMB_DIM = 64
RANK = 16
SCALING = 32.0 / 16.0
NUM_CORES = 2
NUM_SUBCORES = 16
NW = NUM_CORES * NUM_SUBCORES
CHUNK = 128


def _make_lora_embed(n_idx):
  assert n_idx % (NW * CHUNK) == 0
  per_w = n_idx // NW
  n_step = per_w // CHUNK
  mesh = plsc.VectorSubcoreMesh(core_axis_name="c", subcore_axis_name="s")

  @functools.partial(
      pl.kernel,
      out_type=jax.ShapeDtypeStruct((n_idx, EMB_DIM), jnp.float32),
      mesh=mesh,
      scratch_types=[
          pltpu.VMEM((CHUNK,), jnp.int32),
          pltpu.VMEM((CHUNK, EMB_DIM), jnp.float32),
          pltpu.VMEM((CHUNK, RANK), jnp.float32),
          pltpu.VMEM((RANK, EMB_DIM), jnp.float32),
          pltpu.SemaphoreType.DMA,
          pltpu.SemaphoreType.DMA,
      ],
      compiler_params=pltpu.CompilerParams(use_tc_tiling_on_sc=False),
  )
  def lora_embed(x_hbm, w_hbm, a_hbm, b_hbm, out_hbm,
                 idx_v, rows_v, lora_v, bmat_v, sem0, sem1):
    wid = lax.axis_index("s") * NUM_CORES + lax.axis_index("c")
    base = wid * per_w
    pltpu.sync_copy(b_hbm, bmat_v)

    def step(i, carry):
      off = base + i * CHUNK
      pltpu.sync_copy(x_hbm.at[pl.ds(off, CHUNK)], idx_v)
      cw = pltpu.async_copy(w_hbm.at[idx_v], rows_v, sem0)
      ca = pltpu.async_copy(a_hbm.at[idx_v], lora_v, sem1)
      cw.wait()
      ca.wait()

      def row(c, carry2):
        acc = [rows_v[c, pl.ds(16 * j, 16)] for j in range(4)]
        lr = lora_v[c, pl.ds(0, RANK)]
        for k in range(RANK):
          s = lr[k]
          for j in range(4):
            acc[j] = acc[j] + s * bmat_v[k, pl.ds(16 * j, 16)]
        for j in range(4):
          rows_v[c, pl.ds(16 * j, 16)] = acc[j]
        return carry2

      lax.fori_loop(0, CHUNK, row, 0, unroll=False)
      pltpu.sync_copy(rows_v, out_hbm.at[pl.ds(off, CHUNK)])
      return carry

    lax.fori_loop(0, n_step, step, 0, unroll=False)

  return lora_embed


def kernel(x, weight, lora_A, lora_B):
  b, l = x.shape
  xf = x.reshape(-1).astype(jnp.int32)
  b_scaled = (SCALING * lora_B).astype(jnp.float32)
  out = _make_lora_embed(xf.shape[0])(xf, weight, lora_A, b_scaled)
  return out.reshape(b, l, EMB_DIM)


# speedup vs baseline: 1.1189x; 1.1189x over previous
import functools

import jax
import jax.numpy as jnp
from jax import lax
from jax.experimental import pallas as pl
from jax.experimental.pallas import tpu as pltpu
from jax.experimental.pallas import tpu_sc as plsc

EMB_DIM = 64
RANK = 16
SCALING = 32.0 / 16.0
NUM_CORES = 2
NUM_SUBCORES = 16
NW = NUM_CORES * NUM_SUBCORES
CHUNK = 128


def _make_lora_embed(n_idx):
  assert n_idx % (NW * 2 * CHUNK) == 0
  per_w = n_idx // NW
  n_step = per_w // CHUNK
  mesh = plsc.VectorSubcoreMesh(core_axis_name="c", subcore_axis_name="s")

  @functools.partial(
      pl.kernel,
      out_type=jax.ShapeDtypeStruct((n_idx, EMB_DIM), jnp.float32),
      mesh=mesh,
      scratch_types=[
          pltpu.VMEM((n_step, CHUNK), jnp.int32),
          pltpu.VMEM((2, CHUNK, EMB_DIM), jnp.float32),
          pltpu.VMEM((2, CHUNK, RANK), jnp.float32),
          pltpu.VMEM((RANK, EMB_DIM), jnp.float32),
          pltpu.SemaphoreType.DMA,
          pltpu.SemaphoreType.DMA,
          pltpu.SemaphoreType.DMA,
          pltpu.SemaphoreType.DMA,
          pltpu.SemaphoreType.DMA,
          pltpu.SemaphoreType.DMA,
      ],
      compiler_params=pltpu.CompilerParams(use_tc_tiling_on_sc=False),
  )
  def lora_embed(x_hbm, w_hbm, a_hbm, b_hbm, out_hbm,
                 idx_all, rows_v, lora_v, bmat_v,
                 gw0, gw1, ga0, ga1, ws0, ws1):
    gw = (gw0, gw1)
    ga = (ga0, ga1)
    ws = (ws0, ws1)
    wid = lax.axis_index("s") * NUM_CORES + lax.axis_index("c")
    base = wid * per_w
    pltpu.sync_copy(b_hbm, bmat_v)
    pltpu.sync_copy(x_hbm.at[wid], idx_all)

    def issue(i, s):
      pltpu.async_copy(w_hbm.at[idx_all.at[i]], rows_v.at[s], gw[s])
      pltpu.async_copy(a_hbm.at[idx_all.at[i]], lora_v.at[s], ga[s])

    def drain_gathers(s):
      pltpu.make_async_copy(w_hbm.at[pl.ds(0, CHUNK)], rows_v.at[s], gw[s]).wait()
      pltpu.make_async_copy(a_hbm.at[pl.ds(0, CHUNK)], lora_v.at[s], ga[s]).wait()

    def drain_write(s):
      pltpu.make_async_copy(
          rows_v.at[s], out_hbm.at[pl.ds(0, CHUNK)], ws[s]).wait()

    bregs = [
        tuple(bmat_v[k, pl.ds(16 * j, 16)] for k in range(8 * h, 8 * h + 8)
              for j in range(4))
        for h in range(2)
    ]

    def make_pass(s, h):
      def row(c, bcarry):
        lr = lora_v[s, c, pl.ds(0, RANK)]
        acc = [rows_v[s, c, pl.ds(16 * j, 16)] for j in range(4)]
        acc2 = [None] * 4
        for kk in range(8):
          k = 8 * h + kk
          sc = lr[k]
          for j in range(4):
            t = sc * bcarry[4 * kk + j]
            if kk == 0:
              acc[j] = acc[j] + t
            elif kk == 1:
              acc2[j] = t
            elif kk % 2 == 0:
              acc[j] = acc[j] + t
            else:
              acc2[j] = acc2[j] + t
        for j in range(4):
          rows_v[s, c, pl.ds(16 * j, 16)] = acc[j] + acc2[j]
        return bcarry

      return row

    def compute(s):
      for h in range(2):
        lax.fori_loop(0, CHUNK, make_pass(s, h), bregs[h], unroll=False)

    issue(0, 0)

    def outer(g, carry):
      for b in range(2):
        i = 2 * g + b
        nxt = i + 1

        @pl.when(nxt < n_step)
        def _():
          if b == 1:
            drain_write(0)
          else:

            @pl.when(i >= 1)
            def _():
              drain_write(1)

          issue(nxt, 1 - b)

        drain_gathers(b)
        compute(b)
        off = base + i * CHUNK
        pltpu.async_copy(rows_v.at[b], out_hbm.at[pl.ds(off, CHUNK)], ws[b])
      return carry

    lax.fori_loop(0, n_step // 2, outer, 0, unroll=False)
    drain_write(0)
    drain_write(1)

  return lora_embed


def kernel(x, weight, lora_A, lora_B):
  b, l = x.shape
  n = b * l
  xf = x.reshape(NW, n // (NW * CHUNK), CHUNK).astype(jnp.int32)
  b_scaled = (SCALING * lora_B).astype(jnp.float32)
  out = _make_lora_embed(n)(xf, weight, lora_A, b_scaled)
  return out.reshape(b, l, EMB_DIM)
